# Initial kernel scaffold; baseline (speedup 1.0000x reference)
import jax
import jax.numpy as jnp
from jax.experimental import pallas as pl


def kernel(x, edge_index, edge_type, batch, rel_labels, drop_prob, rel_table, rel_graph_emb, W1, W1_loop, W1_rel, loop_rel1, W2, W2_loop, W2_rel, loop_rel2, W3, W3_loop, W3_rel, loop_rel3, lin_W, lin_b):
    raise NotImplementedError("write your pallas kernel here")


# speedup vs baseline: 2.8659x; 2.8659x over previous
import functools

import jax
import jax.numpy as jnp
from jax import lax
from jax.experimental import pallas as pl
from jax.experimental.pallas import tpu as pltpu
from jax.experimental.pallas import tpu_sc as plsc

_N = 10000
_D = 128
_NT = 16
_NG = 128
_NC = 2
_NS = 16
_NW = _NC * _NS
_CH = 128
_RPT = 640
_ACC = _RPT * _NS
_RPW = _N // _NS

_HIGH = lax.Precision.HIGHEST

def _mesh():
    return plsc.VectorSubcoreMesh(
        core_axis_name="c", subcore_axis_name="s",
        num_cores=_NC, num_subcores=_NS)


def _gather_scatter_body(nchunk, width, table_hbm, src_hbm, dst_hbm, zeros_hbm,
                         out_hbm, src_v, dst_v, rows_v, acc_sh, sem):
    c = lax.axis_index("c")
    s = lax.axis_index("s")
    wid = s * _NC + c
    pltpu.sync_copy(src_hbm.at[wid], src_v)
    pltpu.sync_copy(dst_hbm.at[wid], dst_v)
    pltpu.sync_copy(zeros_hbm, acc_sh.at[pl.ds(s * _RPT, _RPT)])
    plsc.subcore_barrier()

    def body(i, carry):
        pltpu.async_copy(table_hbm.at[src_v.at[i]], rows_v, sem).wait()
        pltpu.sync_copy(rows_v, acc_sh.at[dst_v.at[i]], add=True)
        return carry

    lax.fori_loop(0, nchunk, body, 0)
    plsc.subcore_barrier()
    pltpu.sync_copy(acc_sh.at[pl.ds(s * _RPT, _RPT)],
                    out_hbm.at[c, pl.ds(s * _RPT, _RPT)])


def _make_gs_kernel(nchunk, width):
    return pl.kernel(
        functools.partial(_gather_scatter_body, nchunk, width),
        out_type=jax.ShapeDtypeStruct((_NC, _ACC, width), jnp.float32),
        mesh=_mesh(),
        scratch_types=[
            pltpu.VMEM((nchunk, _CH), jnp.int32),
            pltpu.VMEM((nchunk, _CH), jnp.int32),
            pltpu.VMEM((_CH, width), jnp.float32),
            pltpu.VMEM_SHARED((_ACC, width), jnp.float32),
            pltpu.SemaphoreType.DMA,
        ],
    )


def _stage_body(nchain, relu, S_ref, hist_ref, h_ref, g_ref, *rest):
    chain = rest[:nchain]
    W_ref, Wl_ref, lr_ref, o_ref = rest[nchain:]
    g = g_ref[...]
    for Wr in chain:
        g = lax.dot(g, Wr[...], precision=_HIGH)
    R = jnp.concatenate([g, -g], axis=0)
    S = S_ref[0] + S_ref[1]
    hist = hist_ref[0][:, :_NT] + hist_ref[1][:, :_NT]
    deg = jnp.maximum(jnp.sum(hist, axis=1, keepdims=True), 1.0)
    pre = S - lax.dot(hist, R, precision=_HIGH)
    agg = lax.dot(pre, W_ref[...], precision=_HIGH) / deg
    loop = lax.dot(h_ref[...] - lr_ref[...], Wl_ref[...], precision=_HIGH)
    out = agg + loop
    o_ref[...] = jnp.maximum(out, 0.0) if relu else out


def _make_stage(nchain, relu, blk):
    ngrid = _N // blk
    full = lambda shape: pl.BlockSpec(shape, lambda i: (0,) * len(shape))
    in_specs = [
        pl.BlockSpec((_NC, blk, _D), lambda i: (0, i, 0)),
        pl.BlockSpec((_NC, blk, _D), lambda i: (0, i, 0)),
        pl.BlockSpec((blk, _D), lambda i: (i, 0)),
        full((8, _D)),
    ]
    in_specs += [full((_D, _D))] * nchain
    in_specs += [full((_D, _D)), full((_D, _D)), full((1, _D))]
    return pl.pallas_call(
        functools.partial(_stage_body, nchain, relu),
        grid=(ngrid,),
        in_specs=in_specs,
        out_specs=pl.BlockSpec((blk, _D), lambda i: (i, 0)),
        out_shape=jax.ShapeDtypeStruct((_N, _D), jnp.float32),
    )


def _pool_body(blk, h_ref, b_ref, rl_ref, rt_ref, lw_ref, lb_ref, o_ref,
               ps_ref, cnt_ref):
    i = pl.program_id(0)

    @pl.when(i == 0)
    def _init():
        ps_ref[...] = jnp.zeros_like(ps_ref)
        cnt_ref[...] = jnp.zeros_like(cnt_ref)

    b = b_ref[0]
    onehot = (jnp.broadcast_to(b, (_NG, blk)) ==
              lax.broadcasted_iota(jnp.int32, (_NG, blk), 0)).astype(jnp.float32)
    ps_ref[...] += lax.dot(onehot, h_ref[...], precision=_HIGH)
    cnt_ref[:, 0:1] += jnp.sum(onehot, axis=1, keepdims=True)

    @pl.when(i == pl.num_programs(0) - 1)
    def _fin():
        cnt = jnp.maximum(cnt_ref[:, 0:1], 1.0)
        pooled = ps_ref[...] / cnt
        rl = rl_ref[...]
        oh16 = (jnp.broadcast_to(rl, (_NT, _NG)) ==
                lax.broadcasted_iota(jnp.int32, (_NT, _NG), 0)).astype(jnp.float32)
        relemb = lax.dot_general(oh16, rt_ref[...],
                                 (((0,), (0,)), ((), ())), precision=_HIGH)
        z = (lax.dot(pooled, lw_ref[0:_D, :], precision=_HIGH) +
             lax.dot(relemb, lw_ref[_D:2 * _D, :], precision=_HIGH))
        o_ref[...] = z + lb_ref[...]


def _make_pool(blk):
    ngrid = _N // blk
    full = lambda shape: pl.BlockSpec(shape, lambda i: (0,) * len(shape))
    return pl.pallas_call(
        functools.partial(_pool_body, blk),
        grid=(ngrid,),
        in_specs=[
            pl.BlockSpec((blk, _D), lambda i: (i, 0)),
            pl.BlockSpec((1, 1, blk), lambda i: (i, 0, 0)),
            full((1, _NG)),
            full((_NT, _D)),
            full((2 * _D, _NG)),
            full((1, _NG)),
        ],
        out_specs=full((_NG, _NG)),
        out_shape=jax.ShapeDtypeStruct((_NG, _NG), jnp.float32),
        scratch_shapes=[
            pltpu.VMEM((_NG, _D), jnp.float32),
            pltpu.VMEM((_NG, 8), jnp.float32),
        ],
    )


_BLK = 2000


def kernel(x, edge_index, edge_type, batch, rel_labels, drop_prob, rel_table,
           rel_graph_emb, W1, W1_loop, W1_rel, loop_rel1,
           W2, W2_loop, W2_rel, loop_rel2,
           W3, W3_loop, W3_rel, loop_rel3, lin_W, lin_b):
    E = edge_index.shape[1]
    nchunk = -(-E // (_NW * _CH))
    epad = nchunk * _NW * _CH
    pad = epad - E
    src = jnp.concatenate([edge_index[0], jnp.zeros((pad,), jnp.int32)])
    dst = jnp.concatenate([edge_index[1], jnp.full((pad,), _N, jnp.int32)])
    et = jnp.concatenate([edge_type, jnp.zeros((pad,), jnp.int32)])
    src3 = src.reshape(_NW, nchunk, _CH)
    dst3 = dst.reshape(_NW, nchunk, _CH)
    et3 = et.reshape(_NW, nchunk, _CH)

    eye_pad = jnp.pad(jnp.eye(_NT, dtype=jnp.float32),
                      ((0, 0), (0, _D - _NT)))
    zeros_d = jnp.zeros((_RPT, _D), jnp.float32)

    gs_d = _make_gs_kernel(nchunk, _D)

    hist = gs_d(eye_pad, et3, dst3, zeros_d)
    S1 = gs_d(x, src3, dst3, zeros_d)

    st1 = _make_stage(0, True, _BLK)
    st2 = _make_stage(1, True, _BLK)
    st3 = _make_stage(2, False, _BLK)

    lr1 = loop_rel1.reshape(1, _D)
    lr2 = loop_rel2.reshape(1, _D)
    lr3 = loop_rel3.reshape(1, _D)

    h1 = st1(S1, hist, x, rel_graph_emb, W1, W1_loop, lr1)
    S2 = gs_d(h1, src3, dst3, zeros_d)
    h2 = st2(S2, hist, h1, rel_graph_emb, W1_rel, W2, W2_loop, lr2)
    S3 = gs_d(h2, src3, dst3, zeros_d)
    h3 = st3(S3, hist, h2, rel_graph_emb, W1_rel, W2_rel, W3, W3_loop, lr3)

    batch3 = batch.reshape(_N // _BLK, 1, _BLK)
    rl2 = rel_labels.reshape(1, _NG)
    lwp = jnp.pad(lin_W, ((0, 0), (0, _NG - lin_W.shape[1])))
    lbp = jnp.pad(lin_b, (0, _NG - lin_b.shape[0])).reshape(1, _NG)

    z = _make_pool(_BLK)(h3, batch3, rl2, rel_table, lwp, lbp)
    return z[:, :lin_W.shape[1]]
